# baseline (speedup 1.0000x reference)
import functools

import jax
import jax.numpy as jnp
from jax import lax
from jax.experimental import pallas as pl
from jax.experimental.pallas import tpu as pltpu
from jax.experimental.pallas import tpu_sc as plsc

EMBED = 64
BATCH = 16384
CHUNK = 128
TC_BLK = 2048


def _select_concat_body(pi_ref, pl_ref, pa_ref, a_ref, b_ref, c_ref, out_ref):
    def pick(parity_ref, data_ref):
        odd = (parity_ref[...] & 1) == 1
        return jnp.where(odd, data_ref[:, EMBED:2 * EMBED], data_ref[:, 0:EMBED])

    out_ref[:, 0:EMBED] = pick(pi_ref, a_ref)
    out_ref[:, EMBED:2 * EMBED] = pick(pl_ref, b_ref)
    out_ref[:, 2 * EMBED:3 * EMBED] = pick(pa_ref, c_ref)


def kernel(user_ids, user_locations, user_ages, id_table, location_table, age_table):
    info = plsc.get_sparse_core_info()
    nw = info.num_cores * info.num_subcores
    bpw = BATCH // nw
    nch = bpw // CHUNK

    ids32 = user_ids.astype(jnp.int32)
    locs32 = user_locations.astype(jnp.int32)
    ages32 = user_ages.astype(jnp.int32)

    half_ids = (ids32 >> 1).reshape(nw, nch, CHUNK)
    half_locs = (locs32 >> 1).reshape(nw, nch, CHUNK)
    half_ages = (ages32 >> 1).reshape(nw, nch, CHUNK)

    idt2 = id_table.reshape(-1, 2 * EMBED)
    loct2 = location_table.reshape(-1, 2 * EMBED)
    aget2 = age_table.reshape(-1, 2 * EMBED)

    mesh = plsc.VectorSubcoreMesh(core_axis_name="c", subcore_axis_name="s")
    pair_rows = jax.ShapeDtypeStruct((BATCH, 2 * EMBED), jnp.float32)

    @functools.partial(
        pl.kernel,
        mesh=mesh,
        out_type=(pair_rows, pair_rows, pair_rows),
        scratch_types=[
            pltpu.VMEM((nch, CHUNK), jnp.int32),
            pltpu.VMEM((nch, CHUNK), jnp.int32),
            pltpu.VMEM((nch, CHUNK), jnp.int32),
            pltpu.VMEM((2, CHUNK, 2 * EMBED), jnp.float32),
            pltpu.VMEM((2, CHUNK, 2 * EMBED), jnp.float32),
            pltpu.VMEM((2, CHUNK, 2 * EMBED), jnp.float32),
            pltpu.SemaphoreType.DMA,
            pltpu.SemaphoreType.DMA,
        ],
    )
    def gather(hids_hbm, hlocs_hbm, hages_hbm, idt_hbm, loct_hbm, aget_hbm,
               o0, o1, o2, idx0, idx1, idx2, b0, b1, b2, gsem, wsem):
        wid = lax.axis_index("s") * info.num_cores + lax.axis_index("c")
        base = wid * bpw
        pltpu.sync_copy(hids_hbm.at[wid], idx0)
        pltpu.sync_copy(hlocs_hbm.at[wid], idx1)
        pltpu.sync_copy(hages_hbm.at[wid], idx2)

        def issue_gathers(j, slot):
            return [
                pltpu.async_copy(idt_hbm.at[idx0.at[j]], b0.at[slot], gsem),
                pltpu.async_copy(loct_hbm.at[idx1.at[j]], b1.at[slot], gsem),
                pltpu.async_copy(aget_hbm.at[idx2.at[j]], b2.at[slot], gsem),
            ]

        def issue_writes(j, slot):
            rows = pl.ds(base + j * CHUNK, CHUNK)
            return [
                pltpu.async_copy(b0.at[slot], o0.at[rows, :], wsem),
                pltpu.async_copy(b1.at[slot], o1.at[rows, :], wsem),
                pltpu.async_copy(b2.at[slot], o2.at[rows, :], wsem),
            ]

        gathers = issue_gathers(0, 0)
        writes = []
        for j in range(nch):
            for c in gathers:
                c.wait()
            if j + 1 < nch:
                if writes:
                    for c in writes:
                        c.wait()
                    writes = []
                gathers = issue_gathers(j + 1, (j + 1) % 2)
            writes += issue_writes(j, j % 2)
        for c in writes:
            c.wait()

    e0, e1, e2 = gather(half_ids, half_locs, half_ages, idt2, loct2, aget2)

    select_concat = pl.pallas_call(
        _select_concat_body,
        grid=(BATCH // TC_BLK,),
        in_specs=[pl.BlockSpec((TC_BLK, 1), lambda i: (i, 0))] * 3
        + [pl.BlockSpec((TC_BLK, 2 * EMBED), lambda i: (i, 0))] * 3,
        out_specs=pl.BlockSpec((TC_BLK, 3 * EMBED), lambda i: (i, 0)),
        out_shape=jax.ShapeDtypeStruct((BATCH, 3 * EMBED), jnp.float32),
    )
    return select_concat(
        ids32.reshape(BATCH, 1), locs32.reshape(BATCH, 1), ages32.reshape(BATCH, 1),
        e0, e1, e2)


# speedup vs baseline: 1.3520x; 1.3520x over previous
import functools

import jax
import jax.numpy as jnp
from jax import lax
from jax.experimental import pallas as pl
from jax.experimental.pallas import tpu as pltpu
from jax.experimental.pallas import tpu_sc as plsc

EMBED = 64
BATCH = 16384
G = 32
TC_BLK = 2048


def _concat_body(a_ref, b_ref, c_ref, out_ref):
    out_ref[:, 0:EMBED] = a_ref[...]
    out_ref[:, EMBED:2 * EMBED] = b_ref[...]
    out_ref[:, 2 * EMBED:3 * EMBED] = c_ref[...]


def kernel(user_ids, user_locations, user_ages, id_table, location_table, age_table):
    info = plsc.get_sparse_core_info()
    nw = info.num_cores * info.num_subcores
    bpw = BATCH // nw
    nck = bpw // G

    ids = user_ids.astype(jnp.int32).reshape(nw, bpw // 128, 128)
    locs = user_locations.astype(jnp.int32).reshape(nw, bpw // 128, 128)
    ages = user_ages.astype(jnp.int32).reshape(nw, bpw // 128, 128)

    mesh = plsc.VectorSubcoreMesh(core_axis_name="c", subcore_axis_name="s")
    col = jax.ShapeDtypeStruct((BATCH, EMBED), jnp.float32)

    @functools.partial(
        pl.kernel,
        mesh=mesh,
        compiler_params=pltpu.CompilerParams(needs_layout_passes=False),
        out_type=(col, col, col),
        scratch_types=[
            pltpu.VMEM((bpw // 128, 128), jnp.int32),
            pltpu.VMEM((bpw // 128, 128), jnp.int32),
            pltpu.VMEM((bpw // 128, 128), jnp.int32),
            pltpu.VMEM((8 * G, EMBED), jnp.float32),
            pltpu.VMEM((8 * G, EMBED), jnp.float32),
            pltpu.VMEM((G, EMBED), jnp.float32),
            pltpu.VMEM((G, EMBED), jnp.float32),
            pltpu.SemaphoreType.DMA,
            pltpu.SemaphoreType.DMA,
            pltpu.SemaphoreType.DMA,
            pltpu.SemaphoreType.DMA,
        ],
    )
    def gather(ids_hbm, locs_hbm, ages_hbm, idt_hbm, loct_hbm, aget_hbm,
               o0, o1, o2, si0, si1, si2, bufa, bufb, oba, obb,
               gsa, gsb, wsa, wsb):
        wid = lax.axis_index("s") * info.num_cores + lax.axis_index("c")
        base = wid * bpw
        pltpu.sync_copy(ids_hbm.at[wid], si0)
        pltpu.sync_copy(locs_hbm.at[wid], si1)
        pltpu.sync_copy(ages_hbm.at[wid], si2)

        bufs = (bufa, bufb)
        obufs = (oba, obb)
        gsems = (gsa, gsb)
        wsems = (wsa, wsb)

        def read_idx(ivmem, flat):
            v = plsc.load_gather(ivmem, [
                jnp.full((16,), flat >> 7, jnp.int32),
                jnp.full((16,), flat & 127, jnp.int32),
            ])
            return v[0]

        def issue(smem, tab, c, buf, gsem):
            def body(i, _):
                idx = read_idx(smem, c * G + i)
                src_row = pl.multiple_of((idx >> 3) * 8, 8)
                dst_row = pl.multiple_of(i * 8, 8)
                pltpu.async_copy(tab.at[pl.ds(src_row, 8), :],
                                 buf.at[pl.ds(dst_row, 8), :], gsem)
                return 0

            lax.fori_loop(0, G, body, 0)

        def select(smem, c, buf, obuf):
            def body(i, _):
                idx = read_idx(smem, c * G + i)
                row = i * 8 + (idx & 7)
                for w in range(EMBED // 16):
                    obuf[i, pl.ds(w * 16, 16)] = buf[row, pl.ds(w * 16, 16)]
                return 0

            lax.fori_loop(0, G, body, 0)

        def drain(sem, shape_ref, hbm_ref):
            pltpu.make_async_copy(hbm_ref.at[pl.ds(0, shape_ref.shape[0]), :],
                                  shape_ref, sem).wait()

        for smem, tab, out in ((si0, idt_hbm, o0), (si1, loct_hbm, o1),
                               (si2, aget_hbm, o2)):
            for c in range(nck):
                p = c % 2
                if c >= 2:
                    drain(wsems[p], obufs[p], tab)
                issue(smem, tab, c, bufs[p], gsems[p])
                if c >= 1:
                    q = (c - 1) % 2
                    drain(gsems[q], bufs[q], tab)
                    select(smem, c - 1, bufs[q], obufs[q])
                    pltpu.async_copy(
                        obufs[q],
                        out.at[pl.ds(base + (c - 1) * G, G), :], wsems[q])
            q = (nck - 1) % 2
            drain(gsems[q], bufs[q], tab)
            select(smem, nck - 1, bufs[q], obufs[q])
            pltpu.async_copy(obufs[q],
                             out.at[pl.ds(base + (nck - 1) * G, G), :], wsems[q])
            drain(wsems[0], obufs[0], tab)
            drain(wsems[1], obufs[1], tab)

    e0, e1, e2 = gather(ids, locs, ages, id_table, location_table, age_table)

    concat = pl.pallas_call(
        _concat_body,
        grid=(BATCH // TC_BLK,),
        in_specs=[pl.BlockSpec((TC_BLK, EMBED), lambda i: (i, 0))] * 3,
        out_specs=pl.BlockSpec((TC_BLK, 3 * EMBED), lambda i: (i, 0)),
        out_shape=jax.ShapeDtypeStruct((BATCH, 3 * EMBED), jnp.float32),
    )
    return concat(e0, e1, e2)


# speedup vs baseline: 1.9330x; 1.4297x over previous
import functools

import jax
import jax.numpy as jnp
from jax import lax
from jax.experimental import pallas as pl
from jax.experimental.pallas import tpu as pltpu
from jax.experimental.pallas import tpu_sc as plsc

EMBED = 64
BATCH = 16384
CHUNK = 128
TC_BLK = 2048
RW = 8192


W = 4096


def _repack_body(lo_ref, hi_ref, out_ref):
    out_ref[:, 0:EMBED] = jnp.transpose(lo_ref[...])
    out_ref[:, EMBED:2 * EMBED] = jnp.transpose(hi_ref[...])


def _repack(table_t):
    n = table_t.shape[1]
    grid = (n + 2 * W - 1) // (2 * W)
    max_blk = (n + W - 1) // W - 1
    return pl.pallas_call(
        _repack_body,
        grid=(grid,),
        in_specs=[
            pl.BlockSpec((EMBED, W), lambda i: (0, jnp.minimum(2 * i, max_blk))),
            pl.BlockSpec((EMBED, W), lambda i: (0, jnp.minimum(2 * i + 1, max_blk))),
        ],
        out_specs=pl.BlockSpec((W, 2 * EMBED), lambda i: (i, 0)),
        out_shape=jax.ShapeDtypeStruct((grid * W, 2 * EMBED), jnp.float32),
    )(table_t, table_t)


def _select_concat_body(pi_ref, pl_ref, pa_ref, a_ref, b_ref, c_ref, out_ref):
    def pick(parity_ref, data_ref):
        odd = (parity_ref[...] & 1) == 1
        return jnp.where(odd, data_ref[:, EMBED:2 * EMBED], data_ref[:, 0:EMBED])

    out_ref[:, 0:EMBED] = pick(pi_ref, a_ref)
    out_ref[:, EMBED:2 * EMBED] = pick(pl_ref, b_ref)
    out_ref[:, 2 * EMBED:3 * EMBED] = pick(pa_ref, c_ref)


def kernel(user_ids, user_locations, user_ages, id_table, location_table, age_table):
    info = plsc.get_sparse_core_info()
    nw = info.num_cores * info.num_subcores
    bpw = BATCH // nw
    nch = bpw // CHUNK

    ids32 = user_ids.astype(jnp.int32)
    locs32 = user_locations.astype(jnp.int32)
    ages32 = user_ages.astype(jnp.int32)

    def pair_index(r):
        c = r % (2 * W)
        return (r // (2 * W) * W + c % W).reshape(nw, nch, CHUNK)

    def high_mask(r):
        return (r % (2 * W) >= W).astype(jnp.int32).reshape(BATCH, 1)

    half_ids = pair_index(ids32)
    half_locs = pair_index(locs32)
    half_ages = pair_index(ages32)

    idt2 = _repack(id_table.T)
    loct2 = _repack(location_table.T)
    aget2 = _repack(age_table.T)

    mesh = plsc.VectorSubcoreMesh(core_axis_name="c", subcore_axis_name="s")
    pair_rows = jax.ShapeDtypeStruct((BATCH, 2 * EMBED), jnp.float32)

    @functools.partial(
        pl.kernel,
        mesh=mesh,
        out_type=(pair_rows, pair_rows, pair_rows),
        scratch_types=[
            pltpu.VMEM((nch, CHUNK), jnp.int32),
            pltpu.VMEM((nch, CHUNK), jnp.int32),
            pltpu.VMEM((nch, CHUNK), jnp.int32),
            pltpu.VMEM((2, CHUNK, 2 * EMBED), jnp.float32),
            pltpu.VMEM((2, CHUNK, 2 * EMBED), jnp.float32),
            pltpu.VMEM((2, CHUNK, 2 * EMBED), jnp.float32),
            pltpu.SemaphoreType.DMA,
            pltpu.SemaphoreType.DMA,
        ],
    )
    def gather(hids_hbm, hlocs_hbm, hages_hbm, idt_hbm, loct_hbm, aget_hbm,
               o0, o1, o2, idx0, idx1, idx2, b0, b1, b2, gsem, wsem):
        wid = lax.axis_index("s") * info.num_cores + lax.axis_index("c")
        base = wid * bpw
        pltpu.sync_copy(hids_hbm.at[wid], idx0)
        pltpu.sync_copy(hlocs_hbm.at[wid], idx1)
        pltpu.sync_copy(hages_hbm.at[wid], idx2)

        def issue_gathers(j, slot):
            return [
                pltpu.async_copy(idt_hbm.at[idx0.at[j]], b0.at[slot], gsem),
                pltpu.async_copy(loct_hbm.at[idx1.at[j]], b1.at[slot], gsem),
                pltpu.async_copy(aget_hbm.at[idx2.at[j]], b2.at[slot], gsem),
            ]

        def issue_writes(j, slot):
            rows = pl.ds(base + j * CHUNK, CHUNK)
            return [
                pltpu.async_copy(b0.at[slot], o0.at[rows, :], wsem),
                pltpu.async_copy(b1.at[slot], o1.at[rows, :], wsem),
                pltpu.async_copy(b2.at[slot], o2.at[rows, :], wsem),
            ]

        gathers = issue_gathers(0, 0)
        writes = []
        for j in range(nch):
            for c in gathers:
                c.wait()
            if j + 1 < nch:
                if writes:
                    for c in writes:
                        c.wait()
                    writes = []
                gathers = issue_gathers(j + 1, (j + 1) % 2)
            writes += issue_writes(j, j % 2)
        for c in writes:
            c.wait()

    e0, e1, e2 = gather(half_ids, half_locs, half_ages, idt2, loct2, aget2)

    select_concat = pl.pallas_call(
        _select_concat_body,
        grid=(BATCH // TC_BLK,),
        in_specs=[pl.BlockSpec((TC_BLK, 1), lambda i: (i, 0))] * 3
        + [pl.BlockSpec((TC_BLK, 2 * EMBED), lambda i: (i, 0))] * 3,
        out_specs=pl.BlockSpec((TC_BLK, 3 * EMBED), lambda i: (i, 0)),
        out_shape=jax.ShapeDtypeStruct((BATCH, 3 * EMBED), jnp.float32),
    )
    return select_concat(
        high_mask(ids32), high_mask(locs32), high_mask(ages32), e0, e1, e2)


# speedup vs baseline: 1.9820x; 1.0253x over previous
import functools

import jax
import jax.numpy as jnp
from jax import lax
from jax.experimental import pallas as pl
from jax.experimental.pallas import tpu as pltpu
from jax.experimental.pallas import tpu_sc as plsc

EMBED = 64
BATCH = 16384
CHUNK = 128
TC_BLK = 2048
W = 4096


def _repack_body(lo_ref, hi_ref, out_ref):
    out_ref[:, 0:EMBED] = jnp.transpose(lo_ref[...])
    out_ref[:, EMBED:2 * EMBED] = jnp.transpose(hi_ref[...])


def _repack(table_t):
    n = table_t.shape[1]
    grid = (n + 2 * W - 1) // (2 * W)
    max_blk = (n + W - 1) // W - 1
    return pl.pallas_call(
        _repack_body,
        grid=(grid,),
        in_specs=[
            pl.BlockSpec((EMBED, W), lambda i: (0, jnp.minimum(2 * i, max_blk))),
            pl.BlockSpec((EMBED, W), lambda i: (0, jnp.minimum(2 * i + 1, max_blk))),
        ],
        out_specs=pl.BlockSpec((W, 2 * EMBED), lambda i: (i, 0)),
        out_shape=jax.ShapeDtypeStruct((grid * W, 2 * EMBED), jnp.float32),
    )(table_t, table_t)


def _select_concat_body(m_ref, a_ref, b_ref, c_ref, out_ref):
    def pick(bit, data_ref):
        odd = (m_ref[...] & bit) == bit
        return jnp.where(odd, data_ref[:, EMBED:2 * EMBED], data_ref[:, 0:EMBED])

    out_ref[:, 0:EMBED] = pick(1, a_ref)
    out_ref[:, EMBED:2 * EMBED] = pick(2, b_ref)
    out_ref[:, 2 * EMBED:3 * EMBED] = pick(4, c_ref)


def _sc_gather(tables, index_arrays, nw, bpw, nch):
    nt = len(tables)
    mesh = plsc.VectorSubcoreMesh(core_axis_name="c", subcore_axis_name="s")
    info = plsc.get_sparse_core_info()
    pair_rows = jax.ShapeDtypeStruct((BATCH, 2 * EMBED), jnp.float32)

    @functools.partial(
        pl.kernel,
        mesh=mesh,
        out_type=(pair_rows,) * nt,
        scratch_types=[pltpu.VMEM((nch, CHUNK), jnp.int32)] * nt
        + [pltpu.VMEM((2, CHUNK, 2 * EMBED), jnp.float32)] * nt
        + [pltpu.SemaphoreType.DMA, pltpu.SemaphoreType.DMA],
    )
    def gather(*refs):
        idx_hbm = refs[:nt]
        tab_hbm = refs[nt:2 * nt]
        outs = refs[2 * nt:3 * nt]
        idx_v = refs[3 * nt:4 * nt]
        bufs = refs[4 * nt:5 * nt]
        gsem, wsem = refs[5 * nt:]
        wid = lax.axis_index("s") * info.num_cores + lax.axis_index("c")
        base = wid * bpw
        for h, v in zip(idx_hbm, idx_v):
            pltpu.sync_copy(h.at[wid], v)

        def issue_gathers(j, slot):
            return [pltpu.async_copy(t.at[v.at[j]], b.at[slot], gsem)
                    for t, v, b in zip(tab_hbm, idx_v, bufs)]

        def issue_writes(j, slot):
            rows = pl.ds(base + j * CHUNK, CHUNK)
            return [pltpu.async_copy(b.at[slot], o.at[rows, :], wsem)
                    for b, o in zip(bufs, outs)]

        gathers = issue_gathers(0, 0)
        writes = []
        for j in range(nch):
            for c in gathers:
                c.wait()
            if j + 1 < nch:
                if writes:
                    for c in writes:
                        c.wait()
                    writes = []
                gathers = issue_gathers(j + 1, (j + 1) % 2)
            writes += issue_writes(j, j % 2)
        for c in writes:
            c.wait()

    return gather(*index_arrays, *tables)


def kernel(user_ids, user_locations, user_ages, id_table, location_table, age_table):
    info = plsc.get_sparse_core_info()
    nw = info.num_cores * info.num_subcores
    bpw = BATCH // nw
    nch = bpw // CHUNK

    ids32 = user_ids.astype(jnp.int32)
    locs32 = user_locations.astype(jnp.int32)
    ages32 = user_ages.astype(jnp.int32)

    def pair_index(r):
        c = r % (2 * W)
        return (r // (2 * W) * W + c % W).reshape(nw, nch, CHUNK)

    def high_bit(r, bit):
        return (r % (2 * W) >= W).astype(jnp.int32) * bit

    mask = (high_bit(ids32, 1) | high_bit(locs32, 2)
            | high_bit(ages32, 4)).reshape(BATCH, 1)

    loct2 = _repack(location_table.T)
    aget2 = _repack(age_table.T)
    (e1, e2) = _sc_gather(
        (loct2, aget2), (pair_index(locs32), pair_index(ages32)), nw, bpw, nch)

    idt2 = _repack(id_table.T)
    (e0,) = _sc_gather((idt2,), (pair_index(ids32),), nw, bpw, nch)

    select_concat = pl.pallas_call(
        _select_concat_body,
        grid=(BATCH // TC_BLK,),
        in_specs=[pl.BlockSpec((TC_BLK, 1), lambda i: (i, 0))]
        + [pl.BlockSpec((TC_BLK, 2 * EMBED), lambda i: (i, 0))] * 3,
        out_specs=pl.BlockSpec((TC_BLK, 3 * EMBED), lambda i: (i, 0)),
        out_shape=jax.ShapeDtypeStruct((BATCH, 3 * EMBED), jnp.float32),
    )
    return select_concat(mask, e0, e1, e2)


# speedup vs baseline: 1.9973x; 1.0077x over previous
import functools

import jax
import jax.numpy as jnp
from jax import lax
from jax.experimental import pallas as pl
from jax.experimental.pallas import tpu as pltpu
from jax.experimental.pallas import tpu_sc as plsc

EMBED = 64
BATCH = 16384
CHUNK = 128
TC_BLK = 2048
W = 8192


def _repack_body(lo_ref, hi_ref, out_ref):
    out_ref[:, 0:EMBED] = jnp.transpose(lo_ref[...])
    out_ref[:, EMBED:2 * EMBED] = jnp.transpose(hi_ref[...])


def _repack(table_t):
    n = table_t.shape[1]
    grid = (n + 2 * W - 1) // (2 * W)
    max_blk = (n + W - 1) // W - 1
    return pl.pallas_call(
        _repack_body,
        grid=(grid,),
        in_specs=[
            pl.BlockSpec((EMBED, W), lambda i: (0, jnp.minimum(2 * i, max_blk))),
            pl.BlockSpec((EMBED, W), lambda i: (0, jnp.minimum(2 * i + 1, max_blk))),
        ],
        out_specs=pl.BlockSpec((W, 2 * EMBED), lambda i: (i, 0)),
        out_shape=jax.ShapeDtypeStruct((grid * W, 2 * EMBED), jnp.float32),
    )(table_t, table_t)


def _select_concat_body(m_ref, a_ref, b_ref, c_ref, out_ref):
    odd = (m_ref[...] & 1) == 1
    out_ref[:, 0:EMBED] = jnp.where(
        odd, a_ref[:, EMBED:2 * EMBED], a_ref[:, 0:EMBED])
    out_ref[:, EMBED:2 * EMBED] = b_ref[...]
    out_ref[:, 2 * EMBED:3 * EMBED] = c_ref[...]


def _sc_gather(tables, index_arrays, out_width, untiled, nw, bpw, nch):
    nt = len(tables)
    mesh = plsc.VectorSubcoreMesh(core_axis_name="c", subcore_axis_name="s")
    info = plsc.get_sparse_core_info()
    out_sds = jax.ShapeDtypeStruct((BATCH, out_width), jnp.float32)
    params = pltpu.CompilerParams(use_tc_tiling_on_sc=not untiled)

    @functools.partial(
        pl.kernel,
        mesh=mesh,
        compiler_params=params,
        out_type=(out_sds,) * nt,
        scratch_types=[pltpu.VMEM((nch, CHUNK), jnp.int32)] * nt
        + [pltpu.VMEM((2, CHUNK, out_width), jnp.float32)] * nt
        + [pltpu.SemaphoreType.DMA, pltpu.SemaphoreType.DMA],
    )
    def gather(*refs):
        idx_hbm = refs[:nt]
        tab_hbm = refs[nt:2 * nt]
        outs = refs[2 * nt:3 * nt]
        idx_v = refs[3 * nt:4 * nt]
        bufs = refs[4 * nt:5 * nt]
        gsem, wsem = refs[5 * nt:]
        wid = lax.axis_index("s") * info.num_cores + lax.axis_index("c")
        base = wid * bpw
        for h, v in zip(idx_hbm, idx_v):
            pltpu.sync_copy(h.at[wid], v)

        def issue_gathers(j, slot):
            return [pltpu.async_copy(t.at[v.at[j]], b.at[slot], gsem)
                    for t, v, b in zip(tab_hbm, idx_v, bufs)]

        def issue_writes(j, slot):
            rows = pl.ds(base + j * CHUNK, CHUNK)
            return [pltpu.async_copy(b.at[slot], o.at[rows, :], wsem)
                    for b, o in zip(bufs, outs)]

        gathers = issue_gathers(0, 0)
        writes = []
        for j in range(nch):
            for c in gathers:
                c.wait()
            if j + 1 < nch:
                if writes:
                    for c in writes:
                        c.wait()
                    writes = []
                gathers = issue_gathers(j + 1, (j + 1) % 2)
            writes += issue_writes(j, j % 2)
        for c in writes:
            c.wait()

    return gather(*index_arrays, *tables)


def kernel(user_ids, user_locations, user_ages, id_table, location_table, age_table):
    info = plsc.get_sparse_core_info()
    nw = info.num_cores * info.num_subcores
    bpw = BATCH // nw
    nch = bpw // CHUNK

    ids32 = user_ids.astype(jnp.int32)
    locs32 = user_locations.astype(jnp.int32)
    ages32 = user_ages.astype(jnp.int32)

    c = ids32 % (2 * W)
    pair_ids = (ids32 // (2 * W) * W + c % W).reshape(nw, nch, CHUNK)
    mask = (c >= W).astype(jnp.int32).reshape(BATCH, 1)

    (e1, e2) = _sc_gather(
        (location_table, age_table),
        (locs32.reshape(nw, nch, CHUNK), ages32.reshape(nw, nch, CHUNK)),
        EMBED, True, nw, bpw, nch)

    idt2 = _repack(id_table.T)
    (e0,) = _sc_gather((idt2,), (pair_ids,), 2 * EMBED, False, nw, bpw, nch)

    select_concat = pl.pallas_call(
        _select_concat_body,
        grid=(BATCH // TC_BLK,),
        in_specs=[pl.BlockSpec((TC_BLK, 1), lambda i: (i, 0)),
                  pl.BlockSpec((TC_BLK, 2 * EMBED), lambda i: (i, 0)),
                  pl.BlockSpec((TC_BLK, EMBED), lambda i: (i, 0)),
                  pl.BlockSpec((TC_BLK, EMBED), lambda i: (i, 0))],
        out_specs=pl.BlockSpec((TC_BLK, 3 * EMBED), lambda i: (i, 0)),
        out_shape=jax.ShapeDtypeStruct((BATCH, 3 * EMBED), jnp.float32),
    )
    return select_concat(mask, e0, e1, e2)


# speedup vs baseline: 2.2282x; 1.1156x over previous
import functools

import jax
import jax.numpy as jnp
from jax import lax
from jax.experimental import pallas as pl
from jax.experimental.pallas import tpu as pltpu
from jax.experimental.pallas import tpu_sc as plsc

EMBED = 64
BATCH = 16384
CHUNK = 128
TC_BLK = 2048
W = 8192


def _repack_body(lo_ref, hi_ref, out_ref):
    out_ref[:, 0:EMBED] = jnp.transpose(lo_ref[...])
    out_ref[:, EMBED:2 * EMBED] = jnp.transpose(hi_ref[...])


def _repack(table_t):
    n = table_t.shape[1]
    grid = (n + 2 * W - 1) // (2 * W)
    max_blk = (n + W - 1) // W - 1
    return pl.pallas_call(
        _repack_body,
        grid=(grid,),
        in_specs=[
            pl.BlockSpec((EMBED, W), lambda i: (0, jnp.minimum(2 * i, max_blk))),
            pl.BlockSpec((EMBED, W), lambda i: (0, jnp.minimum(2 * i + 1, max_blk))),
        ],
        out_specs=pl.BlockSpec((W, 2 * EMBED), lambda i: (i, 0)),
        out_shape=jax.ShapeDtypeStruct((grid * W, 2 * EMBED), jnp.float32),
    )(table_t, table_t)


def _select_concat_body(m_ref, a_ref, b_ref, c_ref, out_ref):
    def pick(bit, data_ref):
        odd = (m_ref[...] & bit) == bit
        sel = jnp.where(odd, data_ref[:, EMBED:2 * EMBED], data_ref[:, 0:EMBED])
        return jnp.transpose(sel)

    out_ref[0:EMBED, :] = pick(1, a_ref)
    out_ref[EMBED:2 * EMBED, :] = pick(2, b_ref)
    out_ref[2 * EMBED:3 * EMBED, :] = pick(4, c_ref)


def _sc_gather(tables, index_arrays, nw, bpw, nch):
    nt = len(tables)
    mesh = plsc.VectorSubcoreMesh(core_axis_name="c", subcore_axis_name="s")
    info = plsc.get_sparse_core_info()
    pair_rows = jax.ShapeDtypeStruct((BATCH, 2 * EMBED), jnp.float32)

    @functools.partial(
        pl.kernel,
        mesh=mesh,
        out_type=(pair_rows,) * nt,
        scratch_types=[pltpu.VMEM((nch, CHUNK), jnp.int32)] * nt
        + [pltpu.VMEM((2, CHUNK, 2 * EMBED), jnp.float32)] * nt
        + [pltpu.SemaphoreType.DMA, pltpu.SemaphoreType.DMA],
    )
    def gather(*refs):
        idx_hbm = refs[:nt]
        tab_hbm = refs[nt:2 * nt]
        outs = refs[2 * nt:3 * nt]
        idx_v = refs[3 * nt:4 * nt]
        bufs = refs[4 * nt:5 * nt]
        gsem, wsem = refs[5 * nt:]
        wid = lax.axis_index("s") * info.num_cores + lax.axis_index("c")
        base = wid * bpw
        for h, v in zip(idx_hbm, idx_v):
            pltpu.sync_copy(h.at[wid], v)

        def issue_gathers(j, slot):
            return [pltpu.async_copy(t.at[v.at[j]], b.at[slot], gsem)
                    for t, v, b in zip(tab_hbm, idx_v, bufs)]

        def issue_writes(j, slot):
            rows = pl.ds(base + j * CHUNK, CHUNK)
            return [pltpu.async_copy(b.at[slot], o.at[rows, :], wsem)
                    for b, o in zip(bufs, outs)]

        gathers = issue_gathers(0, 0)
        writes = []
        for j in range(nch):
            for c in gathers:
                c.wait()
            if j + 1 < nch:
                if writes:
                    for c in writes:
                        c.wait()
                    writes = []
                gathers = issue_gathers(j + 1, (j + 1) % 2)
            writes += issue_writes(j, j % 2)
        for c in writes:
            c.wait()

    return gather(*index_arrays, *tables)


def kernel(user_ids, user_locations, user_ages, id_table, location_table, age_table):
    info = plsc.get_sparse_core_info()
    nw = info.num_cores * info.num_subcores
    bpw = BATCH // nw
    nch = bpw // CHUNK

    ids32 = user_ids.astype(jnp.int32)
    locs32 = user_locations.astype(jnp.int32)
    ages32 = user_ages.astype(jnp.int32)

    def pair_index(r):
        c = r % (2 * W)
        return (r // (2 * W) * W + c % W).reshape(nw, nch, CHUNK)

    def high_bit(r, bit):
        return (r % (2 * W) >= W).astype(jnp.int32) * bit

    mask = (high_bit(ids32, 1) | high_bit(locs32, 2)
            | high_bit(ages32, 4)).reshape(BATCH, 1)

    loct2 = _repack(location_table.T)
    aget2 = _repack(age_table.T)
    (e1, e2) = _sc_gather(
        (loct2, aget2), (pair_index(locs32), pair_index(ages32)), nw, bpw, nch)

    idt2 = _repack(id_table.T)
    (e0,) = _sc_gather((idt2,), (pair_index(ids32),), nw, bpw, nch)

    select_concat = pl.pallas_call(
        _select_concat_body,
        grid=(BATCH // TC_BLK,),
        in_specs=[pl.BlockSpec((TC_BLK, 1), lambda i: (i, 0))]
        + [pl.BlockSpec((TC_BLK, 2 * EMBED), lambda i: (i, 0))] * 3,
        out_specs=pl.BlockSpec((3 * EMBED, TC_BLK), lambda i: (0, i)),
        out_shape=jax.ShapeDtypeStruct((3 * EMBED, BATCH), jnp.float32),
    )
    return select_concat(mask, e0, e1, e2).T


# speedup vs baseline: 2.2363x; 1.0036x over previous
import functools

import jax
import jax.numpy as jnp
from jax import lax
from jax.experimental import pallas as pl
from jax.experimental.pallas import tpu as pltpu
from jax.experimental.pallas import tpu_sc as plsc

EMBED = 64
BATCH = 16384
CHUNK = 128
TC_BLK = 2048
W = 8192


def _repack_body(lo_ref, hi_ref, out_ref):
    out_ref[:, 0:EMBED] = jnp.transpose(lo_ref[...])
    out_ref[:, EMBED:2 * EMBED] = jnp.transpose(hi_ref[...])


def _repack(table_t):
    n = table_t.shape[1]
    grid = (n + 2 * W - 1) // (2 * W)
    max_blk = (n + W - 1) // W - 1
    return pl.pallas_call(
        _repack_body,
        grid=(grid,),
        in_specs=[
            pl.BlockSpec((EMBED, W), lambda i: (0, jnp.minimum(2 * i, max_blk))),
            pl.BlockSpec((EMBED, W), lambda i: (0, jnp.minimum(2 * i + 1, max_blk))),
        ],
        out_specs=pl.BlockSpec((W, 2 * EMBED), lambda i: (i, 0)),
        out_shape=jax.ShapeDtypeStruct((grid * W, 2 * EMBED), jnp.float32),
    )(table_t, table_t)


def _select_concat_body(m_ref, a_ref, b_ref, c_ref, out_ref):
    def pick(bit, data_ref):
        odd = (m_ref[...] & bit) == bit
        sel = jnp.where(odd, data_ref[:, EMBED:2 * EMBED], data_ref[:, 0:EMBED])
        return jnp.transpose(sel)

    out_ref[0:EMBED, :] = pick(1, a_ref)
    out_ref[EMBED:2 * EMBED, :] = pick(2, b_ref)
    out_ref[2 * EMBED:3 * EMBED, :] = pick(4, c_ref)


def _sc_gather(tables, index_arrays, nw, bpw, nch):
    nt = len(tables)
    mesh = plsc.VectorSubcoreMesh(core_axis_name="c", subcore_axis_name="s")
    info = plsc.get_sparse_core_info()
    pair_rows = jax.ShapeDtypeStruct((BATCH, 2 * EMBED), jnp.float32)

    @functools.partial(
        pl.kernel,
        mesh=mesh,
        out_type=(pair_rows,) * nt,
        scratch_types=[pltpu.VMEM((nch, CHUNK), jnp.int32)] * nt
        + [pltpu.VMEM((2, CHUNK, 2 * EMBED), jnp.float32)] * nt
        + [pltpu.SemaphoreType.DMA, pltpu.SemaphoreType.DMA],
    )
    def gather(*refs):
        idx_hbm = refs[:nt]
        tab_hbm = refs[nt:2 * nt]
        outs = refs[2 * nt:3 * nt]
        idx_v = refs[3 * nt:4 * nt]
        bufs = refs[4 * nt:5 * nt]
        gsem, wsem = refs[5 * nt:]
        wid = lax.axis_index("s") * info.num_cores + lax.axis_index("c")
        base = wid * bpw
        for h, v in zip(idx_hbm, idx_v):
            pltpu.sync_copy(h.at[wid], v)

        def issue_gathers(j, slot):
            return [pltpu.async_copy(t.at[v.at[j]], b.at[slot], gsem)
                    for t, v, b in zip(tab_hbm, idx_v, bufs)]

        def issue_writes(j, slot):
            rows = pl.ds(base + j * CHUNK, CHUNK)
            return [pltpu.async_copy(b.at[slot], o.at[rows, :], wsem)
                    for b, o in zip(bufs, outs)]

        gathers = issue_gathers(0, 0)
        writes = []
        for j in range(nch):
            for c in gathers:
                c.wait()
            if j + 1 < nch:
                if writes:
                    for c in writes:
                        c.wait()
                    writes = []
                gathers = issue_gathers(j + 1, (j + 1) % 2)
            writes += issue_writes(j, j % 2)
        for c in writes:
            c.wait()

    return gather(*index_arrays, *tables)


def kernel(user_ids, user_locations, user_ages, id_table, location_table, age_table):
    info = plsc.get_sparse_core_info()
    nw = info.num_cores * info.num_subcores
    bpw = BATCH // nw
    nch = bpw // CHUNK

    ids32 = user_ids.astype(jnp.int32)
    locs32 = user_locations.astype(jnp.int32)
    ages32 = user_ages.astype(jnp.int32)

    def pair_index(r):
        c = r % (2 * W)
        return (r // (2 * W) * W + c % W).reshape(nw, nch, CHUNK)

    def high_bit(r, bit):
        return (r % (2 * W) >= W).astype(jnp.int32) * bit

    mask = (high_bit(ids32, 1) | high_bit(locs32, 2)
            | high_bit(ages32, 4)).reshape(BATCH, 1)

    loct2 = _repack(location_table.T)
    aget2 = _repack(age_table.T)
    idt_gated, loct2, aget2 = lax.optimization_barrier(
        (id_table.T, loct2, aget2))
    (e1, e2) = _sc_gather(
        (loct2, aget2), (pair_index(locs32), pair_index(ages32)), nw, bpw, nch)

    idt2 = _repack(idt_gated)
    (e0,) = _sc_gather((idt2,), (pair_index(ids32),), nw, bpw, nch)

    select_concat = pl.pallas_call(
        _select_concat_body,
        grid=(BATCH // TC_BLK,),
        in_specs=[pl.BlockSpec((TC_BLK, 1), lambda i: (i, 0))]
        + [pl.BlockSpec((TC_BLK, 2 * EMBED), lambda i: (i, 0))] * 3,
        out_specs=pl.BlockSpec((3 * EMBED, TC_BLK), lambda i: (0, i)),
        out_shape=jax.ShapeDtypeStruct((3 * EMBED, BATCH), jnp.float32),
    )
    return select_concat(mask, e0, e1, e2).T


# speedup vs baseline: 2.2808x; 1.0199x over previous
import functools

import jax
import jax.numpy as jnp
from jax import lax
from jax.experimental import pallas as pl
from jax.experimental.pallas import tpu as pltpu
from jax.experimental.pallas import tpu_sc as plsc

EMBED = 64
BATCH = 16384
CHUNK = 128
TC_BLK = 4096
W = 16384


def _repack_body(lo_ref, hi_ref, out_ref):
    out_ref[:, 0:EMBED] = jnp.transpose(lo_ref[...])
    out_ref[:, EMBED:2 * EMBED] = jnp.transpose(hi_ref[...])


def _repack(table_t):
    n = table_t.shape[1]
    grid = (n + 2 * W - 1) // (2 * W)
    max_blk = (n + W - 1) // W - 1
    return pl.pallas_call(
        _repack_body,
        grid=(grid,),
        in_specs=[
            pl.BlockSpec((EMBED, W), lambda i: (0, jnp.minimum(2 * i, max_blk))),
            pl.BlockSpec((EMBED, W), lambda i: (0, jnp.minimum(2 * i + 1, max_blk))),
        ],
        out_specs=pl.BlockSpec((W, 2 * EMBED), lambda i: (i, 0)),
        out_shape=jax.ShapeDtypeStruct((grid * W, 2 * EMBED), jnp.float32),
    )(table_t, table_t)


def _select_concat_body(m_ref, a_ref, b_ref, c_ref, out_ref):
    def pick(bit, data_ref):
        odd = (m_ref[...] & bit) == bit
        sel = jnp.where(odd, data_ref[:, EMBED:2 * EMBED], data_ref[:, 0:EMBED])
        return jnp.transpose(sel)

    out_ref[0:EMBED, :] = pick(1, a_ref)
    out_ref[EMBED:2 * EMBED, :] = pick(2, b_ref)
    out_ref[2 * EMBED:3 * EMBED, :] = pick(4, c_ref)


def _sc_gather(tables, index_arrays, nw, bpw, nch):
    nt = len(tables)
    mesh = plsc.VectorSubcoreMesh(core_axis_name="c", subcore_axis_name="s")
    info = plsc.get_sparse_core_info()
    pair_rows = jax.ShapeDtypeStruct((BATCH, 2 * EMBED), jnp.float32)

    @functools.partial(
        pl.kernel,
        mesh=mesh,
        out_type=(pair_rows,) * nt,
        scratch_types=[pltpu.VMEM((nch, CHUNK), jnp.int32)] * nt
        + [pltpu.VMEM((2, CHUNK, 2 * EMBED), jnp.float32)] * nt
        + [pltpu.SemaphoreType.DMA, pltpu.SemaphoreType.DMA],
    )
    def gather(*refs):
        idx_hbm = refs[:nt]
        tab_hbm = refs[nt:2 * nt]
        outs = refs[2 * nt:3 * nt]
        idx_v = refs[3 * nt:4 * nt]
        bufs = refs[4 * nt:5 * nt]
        gsem, wsem = refs[5 * nt:]
        wid = lax.axis_index("s") * info.num_cores + lax.axis_index("c")
        base = wid * bpw
        for h, v in zip(idx_hbm, idx_v):
            pltpu.sync_copy(h.at[wid], v)

        def issue_gathers(j, slot):
            return [pltpu.async_copy(t.at[v.at[j]], b.at[slot], gsem)
                    for t, v, b in zip(tab_hbm, idx_v, bufs)]

        def issue_writes(j, slot):
            rows = pl.ds(base + j * CHUNK, CHUNK)
            return [pltpu.async_copy(b.at[slot], o.at[rows, :], wsem)
                    for b, o in zip(bufs, outs)]

        gathers = issue_gathers(0, 0)
        writes = []
        for j in range(nch):
            for c in gathers:
                c.wait()
            if j + 1 < nch:
                if writes:
                    for c in writes:
                        c.wait()
                    writes = []
                gathers = issue_gathers(j + 1, (j + 1) % 2)
            writes += issue_writes(j, j % 2)
        for c in writes:
            c.wait()

    return gather(*index_arrays, *tables)


def kernel(user_ids, user_locations, user_ages, id_table, location_table, age_table):
    info = plsc.get_sparse_core_info()
    nw = info.num_cores * info.num_subcores
    bpw = BATCH // nw
    nch = bpw // CHUNK

    ids32 = user_ids.astype(jnp.int32)
    locs32 = user_locations.astype(jnp.int32)
    ages32 = user_ages.astype(jnp.int32)

    def pair_index(r):
        c = r % (2 * W)
        return (r // (2 * W) * W + c % W).reshape(nw, nch, CHUNK)

    def high_bit(r, bit):
        return (r % (2 * W) >= W).astype(jnp.int32) * bit

    mask = (high_bit(ids32, 1) | high_bit(locs32, 2)
            | high_bit(ages32, 4)).reshape(BATCH, 1)

    loct2 = _repack(location_table.T)
    aget2 = _repack(age_table.T)
    idt_gated, loct2, aget2 = lax.optimization_barrier(
        (id_table.T, loct2, aget2))
    (e1, e2) = _sc_gather(
        (loct2, aget2), (pair_index(locs32), pair_index(ages32)), nw, bpw, nch)

    idt2 = _repack(idt_gated)
    (e0,) = _sc_gather((idt2,), (pair_index(ids32),), nw, bpw, nch)

    select_concat = pl.pallas_call(
        _select_concat_body,
        grid=(BATCH // TC_BLK,),
        in_specs=[pl.BlockSpec((TC_BLK, 1), lambda i: (i, 0))]
        + [pl.BlockSpec((TC_BLK, 2 * EMBED), lambda i: (i, 0))] * 3,
        out_specs=pl.BlockSpec((3 * EMBED, TC_BLK), lambda i: (0, i)),
        out_shape=jax.ShapeDtypeStruct((3 * EMBED, BATCH), jnp.float32),
    )
    return select_concat(mask, e0, e1, e2).T
